# Initial kernel scaffold; baseline (speedup 1.0000x reference)
import jax
import jax.numpy as jnp
from jax.experimental import pallas as pl


def kernel(x, label, codes, pred_class, weight, centroids):
    raise NotImplementedError("write your pallas kernel here")


# speedup vs baseline: 4.0473x; 4.0473x over previous
import functools
import math

import jax
import jax.numpy as jnp
from jax import lax
from jax.experimental import pallas as pl
from jax.experimental.pallas import tpu as pltpu
from jax.experimental.pallas import tpu_sc as plsc

_S = 16.0
_M = 0.5
_COS_M = math.cos(_M)
_SIN_M = math.sin(_M)
_TH = math.cos(math.pi - _M)
_MM = math.sin(math.pi - _M) * _M


def _arc_body(x_ref, w_ref, label_ref, out_ref):
    bn = out_ref.shape[1]
    j = pl.program_id(0)
    x = x_ref[...]
    w = w_ref[...]
    xn = jnp.sqrt(jnp.sum(x * x, axis=1, keepdims=True)) + 1e-12
    wn = jnp.sqrt(jnp.sum(w * w, axis=1, keepdims=True)) + 1e-12
    dots = lax.dot_general(x, w, (((1,), (1,)), ((), ())),
                           preferred_element_type=jnp.float32)
    cosine = dots * (1.0 / xn) * (1.0 / wn).T
    sine = jnp.sqrt(jnp.clip(1.0 - cosine * cosine, 0.0, 1.0))
    phi = cosine * _COS_M - sine * _SIN_M
    phi = jnp.where(cosine > _TH, phi, cosine - _MM)
    col = lax.broadcasted_iota(jnp.int32, cosine.shape, 1) + j * bn
    onehot = col == label_ref[...]
    out_ref[...] = jnp.where(onehot, phi, cosine) * _S


def _arcface_logits(x, weight, label):
    B, D = x.shape
    K = weight.shape[0]
    BM, BN = 512, 1024
    grid = (K // BN, B // BM)
    return pl.pallas_call(
        _arc_body,
        grid=grid,
        in_specs=[
            pl.BlockSpec((BM, D), lambda j, i: (i, 0)),
            pl.BlockSpec((BN, D), lambda j, i: (j, 0)),
            pl.BlockSpec((BM, 1), lambda j, i: (i, 0)),
        ],
        out_specs=pl.BlockSpec((BM, BN), lambda j, i: (i, j)),
        out_shape=jax.ShapeDtypeStruct((B, K), jnp.float32),
        compiler_params=pltpu.CompilerParams(
            dimension_semantics=("arbitrary", "arbitrary"),
        ),
    )(x, weight, label.reshape(B, 1).astype(jnp.int32))


def _make_dist_kernel(B, K, KC, D):
    info = plsc.get_sparse_core_info()
    NC, NS, L = info.num_cores, info.num_subcores, info.num_lanes
    NW = NC * NS
    BPW = B // NW
    CB = 16
    NCHUNK = BPW // CB
    ROW = KC * D
    mesh = plsc.VectorSubcoreMesh(core_axis_name="c", subcore_axis_name="s")

    @functools.partial(
        pl.kernel,
        mesh=mesh,
        out_type=jax.ShapeDtypeStruct((B,), jnp.float32),
        scratch_types=[
            pltpu.VMEM((CB,), jnp.int32),
            pltpu.VMEM((CB, ROW), jnp.float32),
            pltpu.VMEM((CB, D), jnp.float32),
            pltpu.VMEM((BPW,), jnp.float32),
            pltpu.SemaphoreType.DMA,
        ],
    )
    def dist_kernel(cents_hbm, codes_hbm, pred_hbm, out_hbm,
                    idx_v, rows_v, codes_v, res_v, sem):
        wid = lax.axis_index("s") * NC + lax.axis_index("c")
        base = wid * BPW

        def chunk_body(ci, _):
            off = base + ci * CB
            pltpu.sync_copy(pred_hbm.at[pl.ds(off, CB)], idx_v)
            pltpu.async_copy(cents_hbm.at[idx_v], rows_v, sem).wait()
            pltpu.sync_copy(codes_hbm.at[pl.ds(off, CB), :], codes_v)

            def row_body(r, res):
                zero = jnp.zeros((L,), jnp.float32)
                lane = lax.iota(jnp.int32, L)

                def s_body(s, accs):
                    cv = codes_v[r, pl.ds(s * L, L)]
                    return tuple(
                        accs[c] + jnp.abs(cv - rows_v[r, pl.ds(c * D + s * L, L)])
                        for c in range(KC))

                accs = lax.fori_loop(0, D // L, s_body, (zero,) * KC)

                dnums = lax.GatherDimensionNumbers(
                    offset_dims=(), collapsed_slice_dims=(0,),
                    start_index_map=(0,))

                def lane_sum(v):
                    for k in (1, 2, 4, 8):
                        perm = lane ^ k
                        v = v + lax.gather(
                            v, perm[:, None], dnums, (1,),
                            mode=lax.GatherScatterMode.PROMISE_IN_BOUNDS)
                    return v

                sums = [lane_sum(a) for a in accs]
                m = sums[0]
                for c in range(1, KC):
                    m = jnp.minimum(m, sums[c])
                return jnp.where(lane == r, m * (1.0 / D), res)

            res = lax.fori_loop(0, CB, row_body, jnp.zeros((L,), jnp.float32))
            res_v[pl.ds(ci * CB, CB)] = res
            return 0

        lax.fori_loop(0, NCHUNK, chunk_body, 0)
        pltpu.sync_copy(res_v, out_hbm.at[pl.ds(base, BPW)])

    return dist_kernel


def kernel(x, label, codes, pred_class, weight, centroids):
    B, D = x.shape
    K, KC, _ = centroids.shape
    logits = _arcface_logits(x, weight, label)
    dist_fn = _make_dist_kernel(B, K, KC, D)
    dist = dist_fn(centroids.reshape(K, KC * D), codes,
                   pred_class.astype(jnp.int32))
    return (logits, dist)


# speedup vs baseline: 4.4784x; 1.1065x over previous
import functools
import math

import jax
import jax.numpy as jnp
from jax import lax
from jax.experimental import pallas as pl
from jax.experimental.pallas import tpu as pltpu
from jax.experimental.pallas import tpu_sc as plsc

_S = 16.0
_M = 0.5
_COS_M = math.cos(_M)
_SIN_M = math.sin(_M)
_TH = math.cos(math.pi - _M)
_MM = math.sin(math.pi - _M) * _M


def _dense_body(x_ref, w_ref, label_ref, fix_ref, out_ref):
    bn = out_ref.shape[1]
    j = pl.program_id(0)
    x = x_ref[...]
    w = w_ref[...]
    rx = lax.rsqrt(jnp.sum(x * x, axis=1, keepdims=True)) * _S
    rw = lax.rsqrt(jnp.sum(w * w, axis=1, keepdims=True))
    dots = lax.dot_general(x, w, (((1,), (1,)), ((), ())),
                           preferred_element_type=jnp.float32)
    logits = dots * rx * rw.T
    col = lax.broadcasted_iota(jnp.int32, logits.shape, 1) + j * bn
    out_ref[...] = jnp.where(col == label_ref[...], fix_ref[...], logits)


def _dense_logits(x, weight, label, fix):
    B, D = x.shape
    K = weight.shape[0]
    BM, BN = 512, 1024
    grid = (K // BN, B // BM)
    return pl.pallas_call(
        _dense_body,
        grid=grid,
        in_specs=[
            pl.BlockSpec((BM, D), lambda j, i: (i, 0)),
            pl.BlockSpec((BN, D), lambda j, i: (j, 0)),
            pl.BlockSpec((BM, 1), lambda j, i: (i, 0)),
            pl.BlockSpec((BM, 1), lambda j, i: (i, 0)),
        ],
        out_specs=pl.BlockSpec((BM, BN), lambda j, i: (i, j)),
        out_shape=jax.ShapeDtypeStruct((B, K), jnp.float32),
        compiler_params=pltpu.CompilerParams(
            dimension_semantics=("arbitrary", "arbitrary"),
        ),
    )(x, weight, label.reshape(B, 1).astype(jnp.int32), fix.reshape(B, 1))


def _rsqrt_nr(t):
    i = lax.bitcast_convert_type(t, jnp.int32)
    y = lax.bitcast_convert_type(
        jnp.int32(0x5F3759DF) - lax.shift_right_logical(i, 1), jnp.float32)
    for _ in range(3):
        y = y * (1.5 - 0.5 * t * y * y)
    return y


def _make_sc_kernel(B, K, KC, D):
    info = plsc.get_sparse_core_info()
    NC, NS, L = info.num_cores, info.num_subcores, info.num_lanes
    NW = NC * NS
    BPW = B // NW
    CB = 16
    NCHUNK = BPW // CB
    ROW = KC * D
    NSL = D // L
    mesh = plsc.VectorSubcoreMesh(core_axis_name="c", subcore_axis_name="s")

    @functools.partial(
        pl.kernel,
        mesh=mesh,
        out_type=(jax.ShapeDtypeStruct((B,), jnp.float32),
                  jax.ShapeDtypeStruct((B,), jnp.float32)),
        scratch_types=[
            pltpu.VMEM((CB,), jnp.int32),
            pltpu.VMEM((CB,), jnp.int32),
            pltpu.VMEM((CB, ROW), jnp.float32),
            pltpu.VMEM((CB, D), jnp.float32),
            pltpu.VMEM((CB, D), jnp.float32),
            pltpu.VMEM((CB, D), jnp.float32),
            pltpu.VMEM((BPW,), jnp.float32),
            pltpu.VMEM((BPW,), jnp.float32),
            pltpu.SemaphoreType.DMA,
        ],
    )
    def sc_kernel(cents_hbm, codes_hbm, pred_hbm, w_hbm, x_hbm, label_hbm,
                  fix_hbm, dist_hbm,
                  pidx_v, lidx_v, rows_v, wrow_v, codes_v, x_v,
                  fix_v, dist_v, sem):
        wid = lax.axis_index("s") * NC + lax.axis_index("c")
        base = wid * BPW
        lane = lax.iota(jnp.int32, L)
        dnums = lax.GatherDimensionNumbers(
            offset_dims=(), collapsed_slice_dims=(0,), start_index_map=(0,))

        def lane_sum(v):
            for k in (1, 2, 4, 8):
                v = v + lax.gather(
                    v, (lane ^ k)[:, None], dnums, (1,),
                    mode=lax.GatherScatterMode.PROMISE_IN_BOUNDS)
            return v

        def chunk_body(ci, _):
            off = base + ci * CB
            pltpu.sync_copy(pred_hbm.at[pl.ds(off, CB)], pidx_v)
            pltpu.sync_copy(label_hbm.at[pl.ds(off, CB)], lidx_v)
            cp1 = pltpu.async_copy(cents_hbm.at[pidx_v], rows_v, sem)
            cp2 = pltpu.async_copy(w_hbm.at[lidx_v], wrow_v, sem)
            pltpu.sync_copy(codes_hbm.at[pl.ds(off, CB), :], codes_v)
            pltpu.sync_copy(x_hbm.at[pl.ds(off, CB), :], x_v)
            cp1.wait()
            cp2.wait()

            def row_body(r, res):
                res_f, res_d = res
                zero = jnp.zeros((L,), jnp.float32)

                def s_body(s, accs):
                    cv = codes_v[r, pl.ds(s * L, L)]
                    xv = x_v[r, pl.ds(s * L, L)]
                    wv = wrow_v[r, pl.ds(s * L, L)]
                    d = tuple(
                        accs[c] + jnp.abs(cv - rows_v[r, pl.ds(c * D + s * L, L)])
                        for c in range(KC))
                    return d + (accs[KC] + xv * wv,
                                accs[KC + 1] + xv * xv,
                                accs[KC + 2] + wv * wv)

                accs = lax.fori_loop(0, NSL, s_body, (zero,) * (KC + 3))
                sums = [lane_sum(a) for a in accs]
                m = sums[0]
                for c in range(1, KC):
                    m = jnp.minimum(m, sums[c])
                dist = m * (1.0 / D)
                dot, sx, sw = sums[KC], sums[KC + 1], sums[KC + 2]
                cos = dot * _rsqrt_nr(jnp.maximum(sx * sw, 1e-30))
                t2 = jnp.clip(1.0 - cos * cos, 0.0, 1.0)
                sine = t2 * _rsqrt_nr(jnp.maximum(t2, 1e-30))
                phi = cos * _COS_M - sine * _SIN_M
                phi = jnp.where(cos > _TH, phi, cos - _MM)
                fix = phi * _S
                return (jnp.where(lane == r, fix, res_f),
                        jnp.where(lane == r, dist, res_d))

            res_f, res_d = lax.fori_loop(
                0, CB, row_body,
                (jnp.zeros((L,), jnp.float32), jnp.zeros((L,), jnp.float32)))
            fix_v[pl.ds(ci * CB, CB)] = res_f
            dist_v[pl.ds(ci * CB, CB)] = res_d
            return 0

        lax.fori_loop(0, NCHUNK, chunk_body, 0)
        pltpu.sync_copy(fix_v, fix_hbm.at[pl.ds(base, BPW)])
        pltpu.sync_copy(dist_v, dist_hbm.at[pl.ds(base, BPW)])

    return sc_kernel


def kernel(x, label, codes, pred_class, weight, centroids):
    B, D = x.shape
    K, KC, _ = centroids.shape
    sc_fn = _make_sc_kernel(B, K, KC, D)
    fix, dist = sc_fn(centroids.reshape(K, KC * D), codes,
                      pred_class.astype(jnp.int32), weight, x,
                      label.astype(jnp.int32))
    logits = _dense_logits(x, weight, label, fix)
    return (logits, dist)


# speedup vs baseline: 6.8476x; 1.5290x over previous
import functools
import math

import jax
import jax.numpy as jnp
from jax import lax
from jax.experimental import pallas as pl
from jax.experimental.pallas import tpu as pltpu
from jax.experimental.pallas import tpu_sc as plsc

_S = 16.0
_M = 0.5
_COS_M = math.cos(_M)
_SIN_M = math.sin(_M)
_TH = math.cos(math.pi - _M)
_MM = math.sin(math.pi - _M) * _M


def _dense_body(x_ref, w_ref, label_ref, out_ref):
    bn = out_ref.shape[1]
    j = pl.program_id(0)
    x = x_ref[...]
    w = w_ref[...]
    rx = lax.rsqrt(jnp.sum(x * x, axis=1, keepdims=True))
    rw = lax.rsqrt(jnp.sum(w * w, axis=1, keepdims=True))
    dots = lax.dot_general(x, w, (((1,), (1,)), ((), ())),
                           preferred_element_type=jnp.float32)
    cosine = dots * rx * rw.T
    col = lax.broadcasted_iota(jnp.int32, cosine.shape, 1) + j * bn
    onehot = col == label_ref[...]
    cos_b = jnp.sum(jnp.where(onehot, cosine, 0.0), axis=1, keepdims=True)
    sine = jnp.sqrt(jnp.clip(1.0 - cos_b * cos_b, 0.0, 1.0))
    phi = cos_b * _COS_M - sine * _SIN_M
    phi = jnp.where(cos_b > _TH, phi, cos_b - _MM)
    out_ref[...] = jnp.where(onehot, phi, cosine) * _S


def _dense_logits(x, weight, label):
    B, D = x.shape
    K = weight.shape[0]
    BM, BN = 512, 1024
    grid = (K // BN, B // BM)
    return pl.pallas_call(
        _dense_body,
        grid=grid,
        in_specs=[
            pl.BlockSpec((BM, D), lambda j, i: (i, 0)),
            pl.BlockSpec((BN, D), lambda j, i: (j, 0)),
            pl.BlockSpec((BM, 1), lambda j, i: (i, 0)),
        ],
        out_specs=pl.BlockSpec((BM, BN), lambda j, i: (i, j)),
        out_shape=jax.ShapeDtypeStruct((B, K), jnp.float32),
        compiler_params=pltpu.CompilerParams(
            dimension_semantics=("arbitrary", "arbitrary"),
        ),
    )(x, weight, label.reshape(B, 1).astype(jnp.int32))


def _make_dist_kernel(B, K, KC, D):
    info = plsc.get_sparse_core_info()
    NC, NS, L = info.num_cores, info.num_subcores, info.num_lanes
    NW = NC * NS
    BPW = B // NW
    CB = 16
    NCHUNK = BPW // CB
    NSL = D // L
    mesh = plsc.VectorSubcoreMesh(core_axis_name="c", subcore_axis_name="s")

    @functools.partial(
        pl.kernel,
        mesh=mesh,
        out_type=jax.ShapeDtypeStruct((B,), jnp.float32),
        scratch_types=[
            pltpu.VMEM((CB,), jnp.int32),
            pltpu.VMEM((CB, KC, D), jnp.float32),
            pltpu.VMEM((CB, D), jnp.float32),
            pltpu.VMEM((BPW,), jnp.float32),
            pltpu.SemaphoreType.DMA,
        ],
    )
    def dist_kernel(cents_hbm, codes_hbm, pred_hbm, out_hbm,
                    idx_v, rows_v, codes_v, res_v, sem):
        wid = lax.axis_index("s") * NC + lax.axis_index("c")
        base = wid * BPW
        lane = lax.iota(jnp.int32, L)
        dnums = lax.GatherDimensionNumbers(
            offset_dims=(), collapsed_slice_dims=(0,), start_index_map=(0,))

        def lane_sum(v):
            for k in (1, 2, 4, 8):
                v = v + lax.gather(
                    v, (lane ^ k)[:, None], dnums, (1,),
                    mode=lax.GatherScatterMode.PROMISE_IN_BOUNDS)
            return v

        def chunk_body(ci, _):
            off = base + ci * CB
            pltpu.sync_copy(pred_hbm.at[pl.ds(off, CB)], idx_v)
            cp = pltpu.async_copy(cents_hbm.at[idx_v], rows_v, sem)
            pltpu.sync_copy(codes_hbm.at[pl.ds(off, CB), :], codes_v)
            cp.wait()

            def row_body(r, res):
                accs = [jnp.zeros((L,), jnp.float32)] * KC
                for s in range(NSL):
                    cv = codes_v[r, pl.ds(s * L, L)]
                    for c in range(KC):
                        accs[c] = accs[c] + jnp.abs(
                            cv - rows_v[r, c, pl.ds(s * L, L)])
                sums = [lane_sum(a) for a in accs]
                m = sums[0]
                for c in range(1, KC):
                    m = jnp.minimum(m, sums[c])
                return jnp.where(lane == r, m * (1.0 / D), res)

            res = lax.fori_loop(0, CB, row_body, jnp.zeros((L,), jnp.float32))
            res_v[pl.ds(ci * CB, CB)] = res
            return 0

        lax.fori_loop(0, NCHUNK, chunk_body, 0)
        pltpu.sync_copy(res_v, out_hbm.at[pl.ds(base, BPW)])

    return dist_kernel


def kernel(x, label, codes, pred_class, weight, centroids):
    B, D = x.shape
    K, KC, _ = centroids.shape
    dist_fn = _make_dist_kernel(B, K, KC, D)
    dist = dist_fn(centroids, codes, pred_class.astype(jnp.int32))
    logits = _dense_logits(x, weight, label)
    return (logits, dist)
